# baseline (speedup 1.0000x reference)
import functools

import jax
import jax.numpy as jnp
from jax import lax
from jax.experimental import pallas as pl
from jax.experimental.pallas import tpu as pltpu
from jax.experimental.pallas import tpu_sc as plsc


def _make_gather2(B, D, n_users, n_items):
    try:
        info = plsc.get_sparse_core_info()
        NC, NS = info.num_cores, info.num_subcores
    except Exception:
        NC, NS = 2, 16
    NW = NC * NS
    assert B % (8 * NW) == 0
    b_per_w = B // NW

    mesh = plsc.VectorSubcoreMesh(core_axis_name="c", subcore_axis_name="s")

    @functools.partial(
        pl.kernel,
        mesh=mesh,
        out_type=[
            jax.ShapeDtypeStruct((B, D), jnp.float32),
            jax.ShapeDtypeStruct((B, D), jnp.float32),
        ],
        scratch_types=[
            pltpu.VMEM((b_per_w,), jnp.int32),
            pltpu.VMEM((b_per_w,), jnp.int32),
            pltpu.VMEM((b_per_w, D), jnp.float32),
            pltpu.VMEM((b_per_w, D), jnp.float32),
            pltpu.SemaphoreType.DMA,
            pltpu.SemaphoreType.DMA,
            pltpu.SemaphoreType.DMA,
            pltpu.SemaphoreType.DMA,
        ],
        compiler_params=pltpu.CompilerParams(use_tc_tiling_on_sc=False),
    )
    def gather2(users_hbm, items_hbm, u_tab_hbm, v_tab_hbm, u_out_hbm,
                v_out_hbm, uidx, iidx, urows, vrows, su, sv, swu, swv):
        wid = lax.axis_index("s") * NC + lax.axis_index("c")
        base = wid * b_per_w
        pltpu.sync_copy(users_hbm.at[pl.ds(base, b_per_w)], uidx)
        cu = pltpu.async_copy(u_tab_hbm.at[uidx], urows, su)
        pltpu.sync_copy(items_hbm.at[pl.ds(base, b_per_w)], iidx)
        cv = pltpu.async_copy(v_tab_hbm.at[iidx], vrows, sv)
        cu.wait()
        cwu = pltpu.async_copy(urows, u_out_hbm.at[pl.ds(base, b_per_w)], swu)
        cv.wait()
        cwv = pltpu.async_copy(vrows, v_out_hbm.at[pl.ds(base, b_per_w)], swv)
        cwu.wait()
        cwv.wait()

    return gather2


def kernel(users, items, U_e, V_e):
    B = users.shape[0]
    D = U_e.shape[1]
    fn = _make_gather2(B, D, U_e.shape[0], V_e.shape[0])
    u_e, v_e = fn(users.astype(jnp.int32), items.astype(jnp.int32), U_e, V_e)
    return (u_e, v_e)
